# speedup vs baseline: 7.4832x; 7.4832x over previous
import functools

import jax
import jax.numpy as jnp
from jax import lax
from jax.experimental import pallas as pl
from jax.experimental.pallas import tpu as pltpu
from jax.experimental.pallas import tpu_sc as plsc

B = 4096
S = 200
E = 128
V = 100000
H = 512
O = 2

NC, NS = 2, 16
NW = NC * NS
BPW = B // NW
CH = S // 2
LANES = 16


def _sc_mean(idx3, table):
    mesh = plsc.VectorSubcoreMesh(core_axis_name="c", subcore_axis_name="s")

    @functools.partial(
        pl.kernel,
        mesh=mesh,
        out_type=jax.ShapeDtypeStruct((B, E), jnp.float32),
        scratch_types=[
            pltpu.VMEM((BPW, 2, CH), jnp.int32),
            pltpu.VMEM((CH, E), jnp.float32),
            pltpu.VMEM((CH, E), jnp.float32),
            pltpu.VMEM((BPW, E), jnp.float32),
            pltpu.SemaphoreType.DMA,
        ],
    )
    def k(table_hbm, idx_hbm, out_hbm, idx_v, rows0, rows1, out_v, sem):
        wid = lax.axis_index("s") * NC + lax.axis_index("c")
        base = wid * BPW
        pltpu.sync_copy(idx_hbm.at[pl.ds(base, BPW)], idx_v)

        @pl.loop(0, BPW)
        def _(b):
            cp0 = pltpu.async_copy(table_hbm.at[idx_v.at[b, 0]], rows0, sem)
            cp1 = pltpu.async_copy(table_hbm.at[idx_v.at[b, 1]], rows1, sem)
            cp0.wait()
            cp1.wait()

            def body(r, accs):
                return tuple(
                    accs[c]
                    + rows0[r, pl.ds(c * LANES, LANES)]
                    + rows1[r, pl.ds(c * LANES, LANES)]
                    for c in range(E // LANES)
                )

            accs = lax.fori_loop(
                0, CH, body,
                tuple(jnp.zeros((LANES,), jnp.float32) for _ in range(E // LANES)),
            )
            for c in range(E // LANES):
                out_v[b, pl.ds(c * LANES, LANES)] = accs[c] * (1.0 / S)

        pltpu.sync_copy(out_v, out_hbm.at[pl.ds(base, BPW)])

    return k(table, idx3)


def _tc_mlp(avg, W1, b1, W2, b2):
    BB = 512

    def body(x_ref, w1_ref, b1_ref, w2_ref, b2_ref, o_ref):
        x = x_ref[...]
        h = jnp.dot(x, w1_ref[...], preferred_element_type=jnp.float32)
        h = jnp.maximum(h + b1_ref[...], 0.0)
        logits = jnp.dot(h, w2_ref[...], preferred_element_type=jnp.float32)
        logits = logits + b2_ref[...]
        m = jnp.max(logits, axis=-1, keepdims=True)
        e = jnp.exp(logits - m)
        lse = m + jnp.log(jnp.sum(e, axis=-1, keepdims=True))
        o_ref[...] = logits - lse

    return pl.pallas_call(
        body,
        grid=(B // BB,),
        in_specs=[
            pl.BlockSpec((BB, E), lambda i: (i, 0)),
            pl.BlockSpec((E, H), lambda i: (0, 0)),
            pl.BlockSpec((1, H), lambda i: (0, 0)),
            pl.BlockSpec((H, O), lambda i: (0, 0)),
            pl.BlockSpec((1, O), lambda i: (0, 0)),
        ],
        out_specs=pl.BlockSpec((BB, O), lambda i: (i, 0)),
        out_shape=jax.ShapeDtypeStruct((B, O), jnp.float32),
    )(avg, W1, b1.reshape(1, H), W2, b2.reshape(1, O))


def kernel(word_indices, table, W1, b1, W2, b2):
    idx3 = word_indices.astype(jnp.int32).reshape(B, 2, CH)
    avg = _sc_mean(idx3, table)
    return _tc_mlp(avg, W1, b1, W2, b2)


# speedup vs baseline: 12.7821x; 1.7081x over previous
import functools

import jax
import jax.numpy as jnp
from jax import lax
from jax.experimental import pallas as pl
from jax.experimental.pallas import tpu as pltpu
from jax.experimental.pallas import tpu_sc as plsc

B = 4096
S = 200
E = 128
V = 100000
H = 512
O = 2

NC, NS = 2, 16
NW = NC * NS
BPW = B // NW
CH0 = 104
CH1 = S - CH0
LANES = 16


def _sc_mean(idx, table):
    mesh = plsc.VectorSubcoreMesh(core_axis_name="c", subcore_axis_name="s")

    @functools.partial(
        pl.kernel,
        mesh=mesh,
        out_type=jax.ShapeDtypeStruct((B, E), jnp.float32),
        scratch_types=[
            pltpu.VMEM((BPW * S,), jnp.int32),
            pltpu.VMEM((CH0, E), jnp.float32),
            pltpu.VMEM((CH1, E), jnp.float32),
            pltpu.VMEM((CH0, E), jnp.float32),
            pltpu.VMEM((CH1, E), jnp.float32),
            pltpu.VMEM((BPW, E), jnp.float32),
            pltpu.SemaphoreType.DMA,
            pltpu.SemaphoreType.DMA,
        ],
    )
    def k(table_hbm, idx_hbm, out_hbm, idx_v, ra0, ra1, rb0, rb1, out_v,
          sem_a, sem_b):
        wid = lax.axis_index("s") * NC + lax.axis_index("c")
        base = wid * BPW
        pltpu.sync_copy(idx_hbm.at[pl.ds(base * S, BPW * S)], idx_v)

        def issue(b, r0, r1, sem):
            off = pl.multiple_of(b * S, 8)
            pltpu.async_copy(table_hbm.at[idx_v.at[pl.ds(off, CH0)]], r0, sem)
            pltpu.async_copy(
                table_hbm.at[idx_v.at[pl.ds(off + CH0, CH1)]], r1, sem)

        def wait(r0, r1, sem):
            pltpu.make_async_copy(table_hbm.at[pl.ds(0, CH0)], r0, sem).wait()
            pltpu.make_async_copy(table_hbm.at[pl.ds(0, CH1)], r1, sem).wait()

        def reduce_into(b, r0, r1):
            def body0(r, accs):
                return tuple(
                    accs[c] + r0[r, pl.ds(c * LANES, LANES)]
                    for c in range(E // LANES)
                )

            def body1(r, accs):
                return tuple(
                    accs[c] + r1[r, pl.ds(c * LANES, LANES)]
                    for c in range(E // LANES)
                )

            accs = tuple(
                jnp.zeros((LANES,), jnp.float32) for _ in range(E // LANES))
            accs = lax.fori_loop(0, CH0, body0, accs)
            accs = lax.fori_loop(0, CH1, body1, accs)
            for c in range(E // LANES):
                out_v[b, pl.ds(c * LANES, LANES)] = accs[c] * (1.0 / S)

        issue(0, ra0, ra1, sem_a)

        @pl.loop(0, BPW, step=2)
        def _(b):
            issue(b + 1, rb0, rb1, sem_b)
            wait(ra0, ra1, sem_a)
            reduce_into(b, ra0, ra1)
            nxt = jnp.where(b + 2 >= BPW, 0, b + 2)
            issue(nxt, ra0, ra1, sem_a)
            wait(rb0, rb1, sem_b)
            reduce_into(b + 1, rb0, rb1)

        wait(ra0, ra1, sem_a)
        pltpu.sync_copy(out_v, out_hbm.at[pl.ds(base, BPW)])

    return k(table, idx)


def _tc_mlp(avg, W1, b1, W2, b2):
    BB = 512

    def body(x_ref, w1_ref, b1_ref, w2_ref, b2_ref, o_ref):
        x = x_ref[...]
        h = jnp.dot(x, w1_ref[...], preferred_element_type=jnp.float32)
        h = jnp.maximum(h + b1_ref[...], 0.0)
        logits = jnp.dot(h, w2_ref[...], preferred_element_type=jnp.float32)
        logits = logits + b2_ref[...]
        m = jnp.max(logits, axis=-1, keepdims=True)
        e = jnp.exp(logits - m)
        lse = m + jnp.log(jnp.sum(e, axis=-1, keepdims=True))
        o_ref[...] = logits - lse

    return pl.pallas_call(
        body,
        grid=(B // BB,),
        in_specs=[
            pl.BlockSpec((BB, E), lambda i: (i, 0)),
            pl.BlockSpec((E, H), lambda i: (0, 0)),
            pl.BlockSpec((1, H), lambda i: (0, 0)),
            pl.BlockSpec((H, O), lambda i: (0, 0)),
            pl.BlockSpec((1, O), lambda i: (0, 0)),
        ],
        out_specs=pl.BlockSpec((BB, O), lambda i: (i, 0)),
        out_shape=jax.ShapeDtypeStruct((B, O), jnp.float32),
    )(avg, W1, b1.reshape(1, H), W2, b2.reshape(1, O))


def kernel(word_indices, table, W1, b1, W2, b2):
    idx = word_indices.astype(jnp.int32).reshape(B * S)
    avg = _sc_mean(idx, table)
    return _tc_mlp(avg, W1, b1, W2, b2)
